# baseline (speedup 1.0000x reference)
import functools

import jax
import jax.numpy as jnp
from jax import lax
from jax.experimental import pallas as pl
from jax.experimental.pallas import tpu as pltpu
from jax.experimental.pallas import tpu_sc as plsc

N = 10000
D = 128
W = 144
NC, NS = 1, 16
NW = NC * NS
CHUNK = 64
NBUF = 4
IB = 8
NBLK = 40
CPT = NBLK * IB
EPT = CPT * CHUNK
E_PAD = NW * EPT
R = 10016
RPT = R // NS


def _sc_agg_body(table, idx, zeros, out,
                 acc, ib0, ib1, rows0, rows1, rows2, rows3,
                 is0, is1, g0, g1, g2, g3, s0, s1, s2, s3, zsem):
    ibs = (ib0, ib1)
    isem = (is0, is1)
    rows = (rows0, rows1, rows2, rows3)
    gsem = (g0, g1, g2, g3)
    ssem = (s0, s1, s2, s3)
    cid = lax.axis_index("c")
    sid = lax.axis_index("s")
    wid = cid * NS + sid

    pltpu.async_copy(zeros, acc.at[pl.ds(sid * RPT, RPT)], zsem)
    pltpu.async_copy(idx.at[wid, pl.ds(0, IB)], ib0, isem[0]).wait()
    pltpu.async_copy(idx.at[wid, pl.ds(IB, IB)], ib1, isem[1])
    for b in range(NBUF):
        pltpu.async_copy(table.at[ib0.at[b, 0]], rows[b], gsem[b])
    pltpu.make_async_copy(zeros, acc.at[pl.ds(sid * RPT, RPT)], zsem).wait()
    plsc.subcore_barrier()

    def outer(j2, carry):
        for jb in range(2):
            j = j2 * 2 + jb
            ib = ibs[jb]
            ibn = ibs[1 - jb]

            @pl.when(j + 1 < NBLK)
            def _():
                pltpu.async_copy(idx.at[wid, pl.ds((j + 1) * IB, IB)], ibn,
                                 isem[1 - jb])

            for k in range(IB):
                c = j * IB + k
                b = k % NBUF
                pltpu.make_async_copy(table.at[ib.at[k, 0]], rows[b],
                                      gsem[b]).wait()
                pltpu.async_copy(rows[b], acc.at[ib.at[k, 1]], ssem[b],
                                 add=True)
                pltpu.make_async_copy(rows[b], acc.at[ib.at[k, 1]],
                                      ssem[b]).wait()
                if k + NBUF < IB:
                    pltpu.async_copy(table.at[ib.at[k + NBUF, 0]], rows[b],
                                     gsem[b])
                else:
                    @pl.when(j + 1 < NBLK)
                    def _():
                        if k == IB - NBUF:
                            pltpu.make_async_copy(
                                idx.at[wid, pl.ds((j + 1) * IB, IB)], ibn,
                                isem[1 - jb]).wait()
                        pltpu.async_copy(
                            table.at[ibn.at[k + NBUF - IB, 0]], rows[b],
                            gsem[b])

        return carry

    lax.fori_loop(0, NBLK // 2, outer, 0)
    plsc.subcore_barrier()

    pltpu.sync_copy(acc.at[pl.ds(sid * RPT, RPT)],
                    out.at[cid, pl.ds(sid * RPT, RPT)])


_SC_SCRATCH = [
    pltpu.VMEM_SHARED((R, W), jnp.float32),
    pltpu.VMEM((IB, 2, CHUNK), jnp.int32),
    pltpu.VMEM((IB, 2, CHUNK), jnp.int32),
] + [pltpu.VMEM((CHUNK, W), jnp.float32)] * NBUF + [
    pltpu.SemaphoreType.DMA] * (2 + 2 * NBUF + 1)


_sc_agg = pl.kernel(
    _sc_agg_body,
    out_type=jax.ShapeDtypeStruct((NC, R, W), jnp.float32),
    mesh=plsc.VectorSubcoreMesh(core_axis_name="c", subcore_axis_name="s",
                                num_cores=NC, num_subcores=NS),
    scratch_types=_SC_SCRATCH,
    compiler_params=pltpu.CompilerParams(use_tc_tiling_on_sc=False),
)


def _aug_cols(nrows):
    col = lax.broadcasted_iota(jnp.int32, (nrows, W - D), 1)
    return jnp.where(col == 0, 1.0, 0.0).astype(jnp.float32)


def _dense_body(aggp, xin, res, wl, bl, wr, g, b, out, *, last):
    acc = aggp[0] + aggp[1]
    ssum = acc[:N, :D]
    cnt = jnp.maximum(acc[:N, D:D + 1], 1.0)
    agg = ssum / cnt
    h = (jnp.dot(agg, wl[...], preferred_element_type=jnp.float32)
         + bl[...][None, :]
         + jnp.dot(xin[...][:, :D], wr[...],
                   preferred_element_type=jnp.float32))
    mu = jnp.mean(h, axis=1, keepdims=True)
    var = jnp.mean((h - mu) * (h - mu), axis=1, keepdims=True)
    hn = (h - mu) * lax.rsqrt(var + 1e-5) * g[...][None, :] + b[...][None, :]
    hr = jnp.maximum(hn, 0.0)
    if last:
        out[...] = hr + res[...]
    else:
        out[...] = jnp.concatenate([hr, _aug_cols(N)], axis=1)


_dense0 = pl.pallas_call(
    functools.partial(_dense_body, last=False),
    out_shape=jax.ShapeDtypeStruct((N, W), jnp.float32),
)

_dense1 = pl.pallas_call(
    functools.partial(_dense_body, last=True),
    out_shape=jax.ShapeDtypeStruct((N, D), jnp.float32),
)


def kernel(x, edge_index, Wl0, bl0, Wr0, g0, b0, Wl1, bl1, Wr1, g1, b1):
    src = edge_index[0]
    dst = edge_index[1]
    pad = E_PAD - src.shape[0]
    srcr = jnp.concatenate(
        [src, jnp.zeros((pad,), jnp.int32)]).reshape(NW, CPT, CHUNK)
    dstr = jnp.concatenate(
        [dst, jnp.full((pad,), N, jnp.int32)]).reshape(NW, CPT, CHUNK)
    idx = jnp.stack([srcr, dstr], axis=2)
    zeros = jnp.zeros((RPT, W), jnp.float32)
    xaug = jnp.concatenate([x, _aug_cols(N)], axis=1)

    agg0 = _sc_agg(xaug, idx, zeros)
    h0aug = _dense0(agg0, xaug, x, Wl0, bl0, Wr0, g0, b0)
    agg1 = _sc_agg(h0aug, idx, zeros)
    return _dense1(agg1, h0aug, x, Wl1, bl1, Wr1, g1, b1)


# speedup vs baseline: 1.0723x; 1.0723x over previous
import functools

import jax
import jax.numpy as jnp
from jax import lax
from jax.experimental import pallas as pl
from jax.experimental.pallas import tpu as pltpu
from jax.experimental.pallas import tpu_sc as plsc

N = 10000
D = 128
W = 144
NC, NS = 2, 16
NW = NC * NS
CHUNK = 64
NBUF = 4
IB = 8
NBLK = 20
CPT = NBLK * IB
EPT = CPT * CHUNK
E_PAD = NW * EPT
R = 10016
RPT = R // NS


def _sc_agg_body(table, idx, zeros, out,
                 acc, ib0, ib1, rows0, rows1, rows2, rows3,
                 is0, is1, g0, g1, g2, g3, s0, s1, s2, s3, zsem):
    ibs = (ib0, ib1)
    isem = (is0, is1)
    rows = (rows0, rows1, rows2, rows3)
    gsem = (g0, g1, g2, g3)
    ssem = (s0, s1, s2, s3)
    cid = lax.axis_index("c")
    sid = lax.axis_index("s")
    wid = cid * NS + sid

    pltpu.async_copy(zeros, acc.at[pl.ds(sid * RPT, RPT)], zsem)
    pltpu.async_copy(idx.at[wid, pl.ds(0, IB)], ib0, isem[0]).wait()
    pltpu.async_copy(idx.at[wid, pl.ds(IB, IB)], ib1, isem[1])
    for b in range(NBUF):
        pltpu.async_copy(table.at[ib0.at[b, 0]], rows[b], gsem[b])
    pltpu.make_async_copy(zeros, acc.at[pl.ds(sid * RPT, RPT)], zsem).wait()
    plsc.subcore_barrier()

    def outer(j2, carry):
        for jb in range(2):
            j = j2 * 2 + jb
            ib = ibs[jb]
            ibn = ibs[1 - jb]

            @pl.when(j + 1 < NBLK)
            def _():
                pltpu.async_copy(idx.at[wid, pl.ds((j + 1) * IB, IB)], ibn,
                                 isem[1 - jb])

            for k in range(IB):
                c = j * IB + k
                b = k % NBUF
                pltpu.make_async_copy(table.at[ib.at[k, 0]], rows[b],
                                      gsem[b]).wait()
                pltpu.async_copy(rows[b], acc.at[ib.at[k, 1]], ssem[b],
                                 add=True)
                pltpu.make_async_copy(rows[b], acc.at[ib.at[k, 1]],
                                      ssem[b]).wait()
                if k + NBUF < IB:
                    pltpu.async_copy(table.at[ib.at[k + NBUF, 0]], rows[b],
                                     gsem[b])
                else:
                    @pl.when(j + 1 < NBLK)
                    def _():
                        if k == IB - NBUF:
                            pltpu.make_async_copy(
                                idx.at[wid, pl.ds((j + 1) * IB, IB)], ibn,
                                isem[1 - jb]).wait()
                        pltpu.async_copy(
                            table.at[ibn.at[k + NBUF - IB, 0]], rows[b],
                            gsem[b])

        return carry

    lax.fori_loop(0, NBLK // 2, outer, 0)
    plsc.subcore_barrier()

    pltpu.sync_copy(acc.at[pl.ds(sid * RPT, RPT)],
                    out.at[cid, pl.ds(sid * RPT, RPT)])


_SC_SCRATCH = [
    pltpu.VMEM_SHARED((R, W), jnp.float32),
    pltpu.VMEM((IB, 2, CHUNK), jnp.int32),
    pltpu.VMEM((IB, 2, CHUNK), jnp.int32),
] + [pltpu.VMEM((CHUNK, W), jnp.float32)] * NBUF + [
    pltpu.SemaphoreType.DMA] * (2 + 2 * NBUF + 1)


_sc_agg = pl.kernel(
    _sc_agg_body,
    out_type=jax.ShapeDtypeStruct((NC, R, W), jnp.float32),
    mesh=plsc.VectorSubcoreMesh(core_axis_name="c", subcore_axis_name="s",
                                num_cores=NC, num_subcores=NS),
    scratch_types=_SC_SCRATCH,
    compiler_params=pltpu.CompilerParams(use_tc_tiling_on_sc=False),
)


def _aug_cols(nrows):
    col = lax.broadcasted_iota(jnp.int32, (nrows, W - D), 1)
    return jnp.where(col == 0, 1.0, 0.0).astype(jnp.float32)


def _dense_body(aggp, xin, res, wl, bl, wr, g, b, out, *, last):
    acc = aggp[0] + aggp[1]
    ssum = acc[:N, :D]
    cnt = jnp.maximum(acc[:N, D:D + 1], 1.0)
    agg = ssum / cnt
    h = (jnp.dot(agg, wl[...], preferred_element_type=jnp.float32)
         + bl[...][None, :]
         + jnp.dot(xin[...][:, :D], wr[...],
                   preferred_element_type=jnp.float32))
    mu = jnp.mean(h, axis=1, keepdims=True)
    var = jnp.mean((h - mu) * (h - mu), axis=1, keepdims=True)
    hn = (h - mu) * lax.rsqrt(var + 1e-5) * g[...][None, :] + b[...][None, :]
    hr = jnp.maximum(hn, 0.0)
    if last:
        out[...] = hr + res[...]
    else:
        out[...] = jnp.concatenate([hr, _aug_cols(N)], axis=1)


_dense0 = pl.pallas_call(
    functools.partial(_dense_body, last=False),
    out_shape=jax.ShapeDtypeStruct((N, W), jnp.float32),
)

_dense1 = pl.pallas_call(
    functools.partial(_dense_body, last=True),
    out_shape=jax.ShapeDtypeStruct((N, D), jnp.float32),
)


def kernel(x, edge_index, Wl0, bl0, Wr0, g0, b0, Wl1, bl1, Wr1, g1, b1):
    src = edge_index[0]
    dst = edge_index[1]
    pad = E_PAD - src.shape[0]
    srcr = jnp.concatenate(
        [src, jnp.zeros((pad,), jnp.int32)]).reshape(NW, CPT, CHUNK)
    dstr = jnp.concatenate(
        [dst, jnp.full((pad,), N, jnp.int32)]).reshape(NW, CPT, CHUNK)
    idx = jnp.stack([srcr, dstr], axis=2)
    zeros = jnp.zeros((RPT, W), jnp.float32)
    xaug = jnp.concatenate([x, _aug_cols(N)], axis=1)

    agg0 = _sc_agg(xaug, idx, zeros)
    h0aug = _dense0(agg0, xaug, x, Wl0, bl0, Wr0, g0, b0)
    agg1 = _sc_agg(h0aug, idx, zeros)
    return _dense1(agg1, h0aug, x, Wl1, bl1, Wr1, g1, b1)


# speedup vs baseline: 1.2242x; 1.1416x over previous
import functools

import jax
import jax.numpy as jnp
from jax import lax
from jax.experimental import pallas as pl
from jax.experimental.pallas import tpu as pltpu
from jax.experimental.pallas import tpu_sc as plsc

N = 10000
D = 128
CW = 16
NC, NS = 2, 16
NW = NC * NS
CHUNK = 64
NBUF = 4
IB = 8
NBLK = 20
CPT = NBLK * IB
EPT = CPT * CHUNK
E_PAD = NW * EPT
R = 10016
RPT = R // NS


def _sc_agg_body(table, idx, zeros, czeros, ones, *refs, counts):
    if counts:
        outs, cout = refs[0], refs[1]
        rest = refs[2:]
        out = None
    else:
        out = refs[0]
        rest = refs[1:]
        outs = cout = None
    (ib0, ib1, rows0, rows1, rows2, rows3,
     is0, is1, g0, g1, g2, g3, s0, s1, s2, s3, zsem) = rest[:17]
    if counts:
        csem = rest[17:17 + NBUF]
        otile = rest[17 + NBUF]
        acc, cacc = rest[18 + NBUF], rest[19 + NBUF]
    else:
        csem = otile = cacc = None
        acc = rest[17]
    ibs = (ib0, ib1)
    isem = (is0, is1)
    rows = (rows0, rows1, rows2, rows3)
    gsem = (g0, g1, g2, g3)
    ssem = (s0, s1, s2, s3)
    cid = lax.axis_index("c")
    sid = lax.axis_index("s")
    wid = cid * NS + sid

    pltpu.async_copy(zeros, acc.at[pl.ds(sid * RPT, RPT)], zsem)
    pltpu.async_copy(idx.at[wid, pl.ds(0, IB)], ib0, isem[0]).wait()
    pltpu.async_copy(idx.at[wid, pl.ds(IB, IB)], ib1, isem[1])
    for b in range(NBUF):
        pltpu.async_copy(table.at[ib0.at[b, 0]], rows[b], gsem[b])
    if counts:
        pltpu.async_copy(czeros, cacc.at[pl.ds(sid * RPT, RPT)], csem[0])
        pltpu.async_copy(ones, otile, csem[1])
        pltpu.make_async_copy(czeros, cacc.at[pl.ds(sid * RPT, RPT)],
                              csem[0]).wait()
        pltpu.make_async_copy(ones, otile, csem[1]).wait()
    pltpu.make_async_copy(zeros, acc.at[pl.ds(sid * RPT, RPT)], zsem).wait()
    plsc.subcore_barrier()

    def outer(j2, carry):
        for jb in range(2):
            j = j2 * 2 + jb
            ib = ibs[jb]
            ibn = ibs[1 - jb]

            @pl.when(j + 1 < NBLK)
            def _():
                pltpu.async_copy(idx.at[wid, pl.ds((j + 1) * IB, IB)], ibn,
                                 isem[1 - jb])

            for k in range(IB):
                b = k % NBUF
                pltpu.make_async_copy(table.at[ib.at[k, 0]], rows[b],
                                      gsem[b]).wait()
                pltpu.async_copy(rows[b], acc.at[ib.at[k, 1]], ssem[b],
                                 add=True)
                if counts:
                    pltpu.async_copy(otile, cacc.at[ib.at[k, 1]], csem[b],
                                     add=True)
                pltpu.make_async_copy(rows[b], acc.at[ib.at[k, 1]],
                                      ssem[b]).wait()
                if counts:
                    pltpu.make_async_copy(otile, cacc.at[ib.at[k, 1]],
                                          csem[b]).wait()
                if k + NBUF < IB:
                    pltpu.async_copy(table.at[ib.at[k + NBUF, 0]], rows[b],
                                     gsem[b])
                else:
                    @pl.when(j + 1 < NBLK)
                    def _():
                        if k == IB - NBUF:
                            pltpu.make_async_copy(
                                idx.at[wid, pl.ds((j + 1) * IB, IB)], ibn,
                                isem[1 - jb]).wait()
                        pltpu.async_copy(
                            table.at[ibn.at[k + NBUF - IB, 0]], rows[b],
                            gsem[b])

        return carry

    lax.fori_loop(0, NBLK // 2, outer, 0)
    plsc.subcore_barrier()

    if counts:
        pltpu.sync_copy(acc.at[pl.ds(sid * RPT, RPT)],
                        outs.at[cid, pl.ds(sid * RPT, RPT)])
        pltpu.sync_copy(cacc.at[pl.ds(sid * RPT, RPT)],
                        cout.at[cid, pl.ds(sid * RPT, RPT)])
    else:
        pltpu.sync_copy(acc.at[pl.ds(sid * RPT, RPT)],
                        out.at[cid, pl.ds(sid * RPT, RPT)])


def _sc_scratch(counts):
    types = [
        pltpu.VMEM((IB, 2, CHUNK), jnp.int32),
        pltpu.VMEM((IB, 2, CHUNK), jnp.int32),
    ] + [pltpu.VMEM((CHUNK, D), jnp.float32)] * NBUF + [
        pltpu.SemaphoreType.DMA] * (2 + 2 * NBUF + 1)
    if counts:
        types += [pltpu.SemaphoreType.DMA] * NBUF
        types += [pltpu.VMEM((CHUNK, CW), jnp.float32)]
        types += [pltpu.VMEM_SHARED((R, D), jnp.float32),
                  pltpu.VMEM_SHARED((R, CW), jnp.float32)]
    else:
        types += [pltpu.VMEM_SHARED((R, D), jnp.float32)]
    return types


_sc_agg0 = pl.kernel(
    functools.partial(_sc_agg_body, counts=True),
    out_type=(jax.ShapeDtypeStruct((NC, R, D), jnp.float32),
              jax.ShapeDtypeStruct((NC, R, CW), jnp.float32)),
    mesh=plsc.VectorSubcoreMesh(core_axis_name="c", subcore_axis_name="s",
                                num_cores=NC, num_subcores=NS),
    scratch_types=_sc_scratch(True),
    compiler_params=pltpu.CompilerParams(use_tc_tiling_on_sc=False),
)

_sc_agg1 = pl.kernel(
    functools.partial(_sc_agg_body, counts=False),
    out_type=jax.ShapeDtypeStruct((NC, R, D), jnp.float32),
    mesh=plsc.VectorSubcoreMesh(core_axis_name="c", subcore_axis_name="s",
                                num_cores=NC, num_subcores=NS),
    scratch_types=_sc_scratch(False),
    compiler_params=pltpu.CompilerParams(use_tc_tiling_on_sc=False),
)


def _dense_body(aggp, cntp, xin, res, wl, bl, wr, g, b, out, *, last):
    acc = aggp[0] + aggp[1]
    cnt = jnp.maximum(cntp[0][:N, :1] + cntp[1][:N, :1], 1.0)
    agg = acc[:N] / cnt
    h = (jnp.dot(agg, wl[...], preferred_element_type=jnp.float32)
         + bl[...][None, :]
         + jnp.dot(xin[...], wr[...], preferred_element_type=jnp.float32))
    mu = jnp.mean(h, axis=1, keepdims=True)
    var = jnp.mean((h - mu) * (h - mu), axis=1, keepdims=True)
    hn = (h - mu) * lax.rsqrt(var + 1e-5) * g[...][None, :] + b[...][None, :]
    hr = jnp.maximum(hn, 0.0)
    if last:
        out[...] = hr + res[...]
    else:
        out[...] = hr


_dense0 = pl.pallas_call(
    functools.partial(_dense_body, last=False),
    out_shape=jax.ShapeDtypeStruct((N, D), jnp.float32),
)

_dense1 = pl.pallas_call(
    functools.partial(_dense_body, last=True),
    out_shape=jax.ShapeDtypeStruct((N, D), jnp.float32),
)


def kernel(x, edge_index, Wl0, bl0, Wr0, g0, b0, Wl1, bl1, Wr1, g1, b1):
    src = edge_index[0]
    dst = edge_index[1]
    pad = E_PAD - src.shape[0]
    srcr = jnp.concatenate(
        [src, jnp.zeros((pad,), jnp.int32)]).reshape(NW, CPT, CHUNK)
    dstr = jnp.concatenate(
        [dst, jnp.full((pad,), N, jnp.int32)]).reshape(NW, CPT, CHUNK)
    idx = jnp.stack([srcr, dstr], axis=2)
    zeros = jnp.zeros((RPT, D), jnp.float32)
    czeros = jnp.zeros((RPT, CW), jnp.float32)
    ones = jnp.concatenate(
        [jnp.ones((CHUNK, 1), jnp.float32),
         jnp.zeros((CHUNK, CW - 1), jnp.float32)], axis=1)

    agg0, cnt = _sc_agg0(x, idx, zeros, czeros, ones)
    h0 = _dense0(agg0, cnt, x, x, Wl0, bl0, Wr0, g0, b0)
    agg1 = _sc_agg1(h0, idx, zeros, czeros, ones)
    return _dense1(agg1, cnt, h0, x, Wl1, bl1, Wr1, g1, b1)


# speedup vs baseline: 1.2377x; 1.0111x over previous
import functools

import jax
import jax.numpy as jnp
from jax import lax
from jax.experimental import pallas as pl
from jax.experimental.pallas import tpu as pltpu
from jax.experimental.pallas import tpu_sc as plsc

N = 10000
D = 128
CW = 16
NC, NS = 2, 16
NW = NC * NS
CHUNK = 128
NBUF = 2
IB = 4
NBLK = 20
CPT = NBLK * IB
EPT = CPT * CHUNK
E_PAD = NW * EPT
R = 10016
RPT = R // NS


def _sc_agg_body(table, idx, zeros, czeros, ones, *refs, counts):
    if counts:
        outs, cout = refs[0], refs[1]
        rest = refs[2:]
        out = None
    else:
        out = refs[0]
        rest = refs[1:]
        outs = cout = None
    ibs = rest[0:2]
    rows = rest[2:2 + NBUF]
    p = 2 + NBUF
    isem = rest[p:p + 2]
    gsem = rest[p + 2:p + 2 + NBUF]
    ssem = rest[p + 2 + NBUF:p + 2 + 2 * NBUF]
    zsem = rest[p + 2 + 2 * NBUF]
    q = p + 3 + 2 * NBUF
    if counts:
        csem = rest[q:q + NBUF]
        otile = rest[q + NBUF]
        acc, cacc = rest[q + NBUF + 1], rest[q + NBUF + 2]
    else:
        csem = otile = cacc = None
        acc = rest[q]
    ib0, ib1 = ibs
    cid = lax.axis_index("c")
    sid = lax.axis_index("s")
    wid = cid * NS + sid

    pltpu.async_copy(zeros, acc.at[pl.ds(sid * RPT, RPT)], zsem)
    pltpu.async_copy(idx.at[wid, pl.ds(0, IB)], ib0, isem[0]).wait()
    pltpu.async_copy(idx.at[wid, pl.ds(IB, IB)], ib1, isem[1])
    for b in range(NBUF):
        pltpu.async_copy(table.at[ib0.at[b, 0]], rows[b], gsem[b])
    if counts:
        pltpu.async_copy(czeros, cacc.at[pl.ds(sid * RPT, RPT)], csem[0])
        pltpu.async_copy(ones, otile, csem[1])
        pltpu.make_async_copy(czeros, cacc.at[pl.ds(sid * RPT, RPT)],
                              csem[0]).wait()
        pltpu.make_async_copy(ones, otile, csem[1]).wait()
    pltpu.make_async_copy(zeros, acc.at[pl.ds(sid * RPT, RPT)], zsem).wait()
    plsc.subcore_barrier()

    def outer(j2, carry):
        for jb in range(2):
            j = j2 * 2 + jb
            ib = ibs[jb]
            ibn = ibs[1 - jb]

            @pl.when(j + 1 < NBLK)
            def _():
                pltpu.async_copy(idx.at[wid, pl.ds((j + 1) * IB, IB)], ibn,
                                 isem[1 - jb])

            for k in range(IB):
                b = k % NBUF
                pltpu.make_async_copy(table.at[ib.at[k, 0]], rows[b],
                                      gsem[b]).wait()
                pltpu.async_copy(rows[b], acc.at[ib.at[k, 1]], ssem[b],
                                 add=True)
                if counts:
                    pltpu.async_copy(otile, cacc.at[ib.at[k, 1]], csem[b],
                                     add=True)
                pltpu.make_async_copy(rows[b], acc.at[ib.at[k, 1]],
                                      ssem[b]).wait()
                if counts:
                    pltpu.make_async_copy(otile, cacc.at[ib.at[k, 1]],
                                          csem[b]).wait()
                if k + NBUF < IB:
                    pltpu.async_copy(table.at[ib.at[k + NBUF, 0]], rows[b],
                                     gsem[b])
                else:
                    @pl.when(j + 1 < NBLK)
                    def _():
                        if k == IB - NBUF:
                            pltpu.make_async_copy(
                                idx.at[wid, pl.ds((j + 1) * IB, IB)], ibn,
                                isem[1 - jb]).wait()
                        pltpu.async_copy(
                            table.at[ibn.at[k + NBUF - IB, 0]], rows[b],
                            gsem[b])

        return carry

    lax.fori_loop(0, NBLK // 2, outer, 0)
    plsc.subcore_barrier()

    if counts:
        pltpu.sync_copy(acc.at[pl.ds(sid * RPT, RPT)],
                        outs.at[cid, pl.ds(sid * RPT, RPT)])
        pltpu.sync_copy(cacc.at[pl.ds(sid * RPT, RPT)],
                        cout.at[cid, pl.ds(sid * RPT, RPT)])
    else:
        pltpu.sync_copy(acc.at[pl.ds(sid * RPT, RPT)],
                        out.at[cid, pl.ds(sid * RPT, RPT)])


def _sc_scratch(counts):
    types = [
        pltpu.VMEM((IB, 2, CHUNK), jnp.int32),
        pltpu.VMEM((IB, 2, CHUNK), jnp.int32),
    ] + [pltpu.VMEM((CHUNK, D), jnp.float32)] * NBUF + [
        pltpu.SemaphoreType.DMA] * (2 + 2 * NBUF + 1)
    if counts:
        types += [pltpu.SemaphoreType.DMA] * NBUF
        types += [pltpu.VMEM((CHUNK, CW), jnp.float32)]
        types += [pltpu.VMEM_SHARED((R, D), jnp.float32),
                  pltpu.VMEM_SHARED((R, CW), jnp.float32)]
    else:
        types += [pltpu.VMEM_SHARED((R, D), jnp.float32)]
    return types


_sc_agg0 = pl.kernel(
    functools.partial(_sc_agg_body, counts=True),
    out_type=(jax.ShapeDtypeStruct((NC, R, D), jnp.float32),
              jax.ShapeDtypeStruct((NC, R, CW), jnp.float32)),
    mesh=plsc.VectorSubcoreMesh(core_axis_name="c", subcore_axis_name="s",
                                num_cores=NC, num_subcores=NS),
    scratch_types=_sc_scratch(True),
    compiler_params=pltpu.CompilerParams(use_tc_tiling_on_sc=False),
)

_sc_agg1 = pl.kernel(
    functools.partial(_sc_agg_body, counts=False),
    out_type=jax.ShapeDtypeStruct((NC, R, D), jnp.float32),
    mesh=plsc.VectorSubcoreMesh(core_axis_name="c", subcore_axis_name="s",
                                num_cores=NC, num_subcores=NS),
    scratch_types=_sc_scratch(False),
    compiler_params=pltpu.CompilerParams(use_tc_tiling_on_sc=False),
)


def _dense_body(aggp, cntp, xin, res, wl, bl, wr, g, b, out, *, last):
    acc = aggp[0] + aggp[1]
    cnt = jnp.maximum(cntp[0][:N, :1] + cntp[1][:N, :1], 1.0)
    agg = acc[:N] / cnt
    h = (jnp.dot(agg, wl[...], preferred_element_type=jnp.float32)
         + bl[...][None, :]
         + jnp.dot(xin[...], wr[...], preferred_element_type=jnp.float32))
    mu = jnp.mean(h, axis=1, keepdims=True)
    var = jnp.mean((h - mu) * (h - mu), axis=1, keepdims=True)
    hn = (h - mu) * lax.rsqrt(var + 1e-5) * g[...][None, :] + b[...][None, :]
    hr = jnp.maximum(hn, 0.0)
    if last:
        out[...] = hr + res[...]
    else:
        out[...] = hr


_dense0 = pl.pallas_call(
    functools.partial(_dense_body, last=False),
    out_shape=jax.ShapeDtypeStruct((N, D), jnp.float32),
)

_dense1 = pl.pallas_call(
    functools.partial(_dense_body, last=True),
    out_shape=jax.ShapeDtypeStruct((N, D), jnp.float32),
)


def kernel(x, edge_index, Wl0, bl0, Wr0, g0, b0, Wl1, bl1, Wr1, g1, b1):
    src = edge_index[0]
    dst = edge_index[1]
    pad = E_PAD - src.shape[0]
    srcr = jnp.concatenate(
        [src, jnp.zeros((pad,), jnp.int32)]).reshape(NW, CPT, CHUNK)
    dstr = jnp.concatenate(
        [dst, jnp.full((pad,), N, jnp.int32)]).reshape(NW, CPT, CHUNK)
    idx = jnp.stack([srcr, dstr], axis=2)
    zeros = jnp.zeros((RPT, D), jnp.float32)
    czeros = jnp.zeros((RPT, CW), jnp.float32)
    ones = jnp.concatenate(
        [jnp.ones((CHUNK, 1), jnp.float32),
         jnp.zeros((CHUNK, CW - 1), jnp.float32)], axis=1)

    agg0, cnt = _sc_agg0(x, idx, zeros, czeros, ones)
    h0 = _dense0(agg0, cnt, x, x, Wl0, bl0, Wr0, g0, b0)
    agg1 = _sc_agg1(h0, idx, zeros, czeros, ones)
    return _dense1(agg1, cnt, h0, x, Wl1, bl1, Wr1, g1, b1)
